# baseline (speedup 1.0000x reference)
import functools

import jax
import jax.numpy as jnp
from jax import lax
from jax.experimental import pallas as pl
from jax.experimental.pallas import tpu as pltpu
from jax.experimental.pallas import tpu_sc as plsc

E = 8
K = 2
D = 1024
FF = 2048
T = 4096
CAP = 1536
DUMMY = E * CAP
BUF_ROWS = DUMMY + 8
BM = 512
NB = CAP // BM

NC, NS = 2, 16
NW = NC * NS
BPW = T // NW
CHUNK = 32

_ROWS = T // K // 2


def _routing_body(te_ref, w_ref, slot_ref, wm_ref, cnt_ref):
    te = te_ref[...]
    li = lax.broadcasted_iota(jnp.int32, (128, 128), 0)
    lj = lax.broadcasted_iota(jnp.int32, (128, 128), 1)
    incl = (li <= lj).astype(jnp.float32)
    ri = lax.broadcasted_iota(jnp.int32, (64, 64), 0)
    rj = lax.broadcasted_iota(jnp.int32, (64, 64), 1)
    strict = (rj < ri).astype(jnp.float32)

    slot = jnp.full((64, 128), DUMMY, jnp.int32)
    cnt_rows = []
    for e in range(E):
        m = te == e
        mf = m.astype(jnp.float32)
        rowinc = jnp.dot(mf, incl, preferred_element_type=jnp.float32)
        rowtot = jnp.sum(mf, axis=1, keepdims=True)
        prev = jnp.dot(strict, rowtot, preferred_element_type=jnp.float32)
        pos = (rowinc - mf + prev).astype(jnp.int32)
        ok = m & (pos < CAP)
        slot = jnp.where(ok, e * CAP + pos, slot)
        cnt_rows.append(jnp.full((1, 128), jnp.sum(mf).astype(jnp.int32),
                                 dtype=jnp.int32))
    slot_ref[...] = slot
    wm_ref[...] = jnp.where(slot != DUMMY, w_ref[...], 0.0)
    cnt_ref[...] = jnp.concatenate(cnt_rows, axis=0)


_routing = pl.pallas_call(
    _routing_body,
    out_shape=(
        jax.ShapeDtypeStruct((64, 128), jnp.int32),
        jax.ShapeDtypeStruct((64, 128), jnp.float32),
        jax.ShapeDtypeStruct((E, 128), jnp.int32),
    ),
)

_NCH_D = BPW // CHUNK
_APW = T * K // NW
_NCH_G = _APW // CHUNK


@functools.lru_cache(maxsize=None)
def _sc_kernels():
    mesh = plsc.VectorSubcoreMesh(core_axis_name="c", subcore_axis_name="s")

    @functools.partial(
        pl.kernel,
        mesh=mesh,
        out_type=jax.ShapeDtypeStruct((BUF_ROWS, D), jnp.float32),
        scratch_types=[
            [pltpu.VMEM((CHUNK,), jnp.int32)] * 2,
            [pltpu.VMEM((CHUNK,), jnp.int32)] * 2,
            [pltpu.VMEM((CHUNK, D), jnp.float32)] * 2,
            [pltpu.SemaphoreType.DMA] * 2,
            [pltpu.SemaphoreType.DMA] * 2,
        ],
    )
    def dispatch(x_hbm, ia_hbm, ib_hbm, buf_hbm,
                 ia_v, ib_v, rows, sem_x, sem_s):
        wid = lax.axis_index("s") * NC + lax.axis_index("c")
        base = wid * BPW

        def start_loads(c, b):
            off = base + c * CHUNK
            return [
                pltpu.async_copy(x_hbm.at[pl.ds(off, CHUNK)], rows[b], sem_x[b]),
                pltpu.async_copy(ia_hbm.at[pl.ds(off, CHUNK)], ia_v[b], sem_x[b]),
                pltpu.async_copy(ib_hbm.at[pl.ds(off, CHUNK)], ib_v[b], sem_x[b]),
            ]

        h_x = [None, None]
        h_s = [[], []]
        for c in range(_NCH_D):
            b = c & 1
            if c == 0:
                h_x[0] = start_loads(0, 0)
            if c + 1 < _NCH_D:
                nb = (c + 1) & 1
                for h in h_s[nb]:
                    h.wait()
                h_s[nb] = []
                h_x[nb] = start_loads(c + 1, nb)
            for h in h_s[b]:
                h.wait()
            h_s[b] = []
            for h in h_x[b]:
                h.wait()
            h_s[b] = [
                pltpu.async_copy(rows[b], buf_hbm.at[ia_v[b]], sem_s[b]),
                pltpu.async_copy(rows[b], buf_hbm.at[ib_v[b]], sem_s[b]),
            ]
        for hs in h_s:
            for h in hs:
                h.wait()

    @functools.partial(
        pl.kernel,
        mesh=mesh,
        out_type=(
            jax.ShapeDtypeStruct((T, D), jnp.float32),
            jax.ShapeDtypeStruct((T, D), jnp.float32),
        ),
        scratch_types=[
            pltpu.VMEM((BPW,), jnp.int32),
            pltpu.VMEM((BPW,), jnp.int32),
            [pltpu.VMEM((CHUNK, D), jnp.float32)] * 2,
            [pltpu.SemaphoreType.DMA] * 2,
            [pltpu.SemaphoreType.DMA] * 2,
        ],
    )
    def gather_back(ob_hbm, ia_hbm, ib_hbm, ya_hbm, yb_hbm,
                    ia_all, ib_all, rows, sem_g, sem_w):
        wid = lax.axis_index("s") * NC + lax.axis_index("c")
        base = wid * BPW
        pltpu.sync_copy(ia_hbm.at[pl.ds(base, BPW)], ia_all)
        pltpu.sync_copy(ib_hbm.at[pl.ds(base, BPW)], ib_all)

        def chunk_args(c):
            idx_all, y_hbm = (ia_all, ya_hbm) if c % 2 == 0 else (ib_all, yb_hbm)
            off = (c // 2) * CHUNK
            return idx_all.at[pl.ds(off, CHUNK)], y_hbm.at[pl.ds(base + off, CHUNK)]

        n = 2 * _NCH_D
        h_g = [None, None]
        h_w = [None, None]
        for c in range(n):
            b = c & 1
            if c == 0:
                src, _ = chunk_args(0)
                h_g[0] = pltpu.async_copy(ob_hbm.at[src], rows[0], sem_g[0])
            if c + 1 < n:
                nb = (c + 1) & 1
                if h_w[nb] is not None:
                    h_w[nb].wait()
                    h_w[nb] = None
                src, _ = chunk_args(c + 1)
                h_g[nb] = pltpu.async_copy(ob_hbm.at[src], rows[nb], sem_g[nb])
            h_g[b].wait()
            _, dst = chunk_args(c)
            h_w[b] = pltpu.async_copy(rows[b], dst, sem_w[b])
        for h in h_w:
            if h is not None:
                h.wait()

    return dispatch, gather_back


NFF = 2


def _gemm_body(cnt_ref, buf_ref, w1_ref, w2_ref, out_ref):
    e = pl.program_id(0)
    f = pl.program_id(1)
    cnt = cnt_ref[e]
    w1 = w1_ref[0]
    w2 = w2_ref[0]
    for rb in range(NB):
        @pl.when(rb * BM < cnt)
        def _():
            a = buf_ref[pl.ds(rb * BM, BM), :].astype(jnp.bfloat16)
            h = jnp.dot(a, w1, preferred_element_type=jnp.float32)
            h = jax.nn.gelu(h, approximate=True)
            o = jnp.dot(h.astype(jnp.bfloat16), w2,
                        preferred_element_type=jnp.float32)
            sl = pl.ds(rb * BM, BM)

            @pl.when(f == 0)
            def _():
                out_ref[sl, :] = o

            @pl.when(f != 0)
            def _():
                out_ref[sl, :] += o


_gemm = pl.pallas_call(
    _gemm_body,
    grid=(E, NFF),
    in_specs=[
        pl.BlockSpec(memory_space=pltpu.MemorySpace.SMEM),
        pl.BlockSpec((CAP, D), lambda e, f: (e, 0)),
        pl.BlockSpec((1, D, FF // NFF), lambda e, f: (e, 0, f)),
        pl.BlockSpec((1, FF // NFF, D), lambda e, f: (e, f, 0)),
    ],
    out_specs=pl.BlockSpec((CAP, D), lambda e, f: (e, 0)),
    out_shape=jax.ShapeDtypeStruct((BUF_ROWS, D), jnp.float32),
    compiler_params=pltpu.CompilerParams(
        dimension_semantics=("parallel", "arbitrary"),
        allow_input_fusion=[False, False, True, True],
        vmem_limit_bytes=100 * 1024 * 1024),
)


def _combine_body(ya_ref, yb_ref, wa_ref, wb_ref, o_ref):
    wa = wa_ref[...]
    wb = wb_ref[...]
    o_ref[...] = (jnp.where(wa != 0.0, wa * ya_ref[...], 0.0)
                  + jnp.where(wb != 0.0, wb * yb_ref[...], 0.0))


_CB = 512
_combine = pl.pallas_call(
    _combine_body,
    grid=(T // _CB,),
    in_specs=[
        pl.BlockSpec((_CB, D), lambda i: (i, 0)),
        pl.BlockSpec((_CB, D), lambda i: (i, 0)),
        pl.BlockSpec((_CB, 1), lambda i: (i, 0)),
        pl.BlockSpec((_CB, 1), lambda i: (i, 0)),
    ],
    out_specs=pl.BlockSpec((_CB, D), lambda i: (i, 0)),
    out_shape=jax.ShapeDtypeStruct((T, D), jnp.float32),
)


def kernel(x, scores, expert_weights, top_experts, W1, W2):
    del scores
    te128 = top_experts.astype(jnp.int32).reshape(64, 128)
    w128 = expert_weights.astype(jnp.float32).reshape(64, 128)
    slot, wm, cnt = _routing(te128, w128)

    slot2 = slot.reshape(T, K)
    ia = slot2[:, 0]
    ib = slot2[:, 1]
    wm2 = wm.reshape(T, K)
    wa = wm2[:, 0:1]
    wb = wm2[:, 1:2]
    counts = cnt[:, 0]

    dispatch, gather_back = _sc_kernels()
    buf = dispatch(x, ia, ib)
    out_buf = _gemm(counts, buf, W1.astype(jnp.bfloat16),
                    W2.astype(jnp.bfloat16))
    ya, yb = gather_back(out_buf, ia, ib)
    return _combine(ya, yb, wa, wb)


# speedup vs baseline: 1.0409x; 1.0409x over previous
import functools

import jax
import jax.numpy as jnp
from jax import lax
from jax.experimental import pallas as pl
from jax.experimental.pallas import tpu as pltpu
from jax.experimental.pallas import tpu_sc as plsc

E = 8
K = 2
D = 1024
FF = 2048
T = 4096
CAP = 1536
DUMMY = E * CAP
BUF_ROWS = DUMMY + 8
BM = 512
NB = CAP // BM

NC, NS = 2, 16
NW = NC * NS
BPW = T // NW
CHUNK = 32

def _routing_body(te_ref, w_ref, slot_ref, wm_ref, cnt_ref):
    te = te_ref[...]
    li = lax.broadcasted_iota(jnp.int32, (128, 128), 0)
    lj = lax.broadcasted_iota(jnp.int32, (128, 128), 1)
    incl = (li <= lj).astype(jnp.float32)
    ri = lax.broadcasted_iota(jnp.int32, (64, 64), 0)
    rj = lax.broadcasted_iota(jnp.int32, (64, 64), 1)
    strict = (rj < ri).astype(jnp.float32)

    slot = jnp.full((64, 128), DUMMY, jnp.int32)
    cnt_rows = []
    for e in range(E):
        m = te == e
        mf = m.astype(jnp.float32)
        rowinc = jnp.dot(mf, incl, preferred_element_type=jnp.float32)
        rowtot = jnp.sum(mf, axis=1, keepdims=True)
        prev = jnp.dot(strict, rowtot, preferred_element_type=jnp.float32)
        pos = (rowinc - mf + prev).astype(jnp.int32)
        ok = m & (pos < CAP)
        slot = jnp.where(ok, e * CAP + pos, slot)
        cnt_rows.append(jnp.full((1, 128), jnp.sum(mf).astype(jnp.int32),
                                 dtype=jnp.int32))
    slot_ref[...] = slot
    wm_ref[...] = jnp.where(slot != DUMMY, w_ref[...], 0.0)
    cnt_ref[...] = jnp.concatenate(cnt_rows, axis=0)


_routing = pl.pallas_call(
    _routing_body,
    out_shape=(
        jax.ShapeDtypeStruct((64, 128), jnp.int32),
        jax.ShapeDtypeStruct((64, 128), jnp.float32),
        jax.ShapeDtypeStruct((E, 128), jnp.int32),
    ),
)

_NCH_D = BPW // CHUNK
_APW = T * K // NW
_NCH_G = _APW // CHUNK


@functools.lru_cache(maxsize=None)
def _sc_kernels():
    mesh = plsc.VectorSubcoreMesh(core_axis_name="c", subcore_axis_name="s")

    @functools.partial(
        pl.kernel,
        mesh=mesh,
        out_type=jax.ShapeDtypeStruct((BUF_ROWS, D), jnp.float32),
        scratch_types=[
            [pltpu.VMEM((CHUNK,), jnp.int32)] * 2,
            [pltpu.VMEM((CHUNK,), jnp.int32)] * 2,
            [pltpu.VMEM((CHUNK, D), jnp.float32)] * 2,
            [pltpu.SemaphoreType.DMA] * 2,
            [pltpu.SemaphoreType.DMA] * 2,
        ],
    )
    def dispatch(x_hbm, ia_hbm, ib_hbm, buf_hbm,
                 ia_v, ib_v, rows, sem_x, sem_s):
        wid = lax.axis_index("s") * NC + lax.axis_index("c")
        base = wid * BPW

        def start_loads(c, b):
            off = base + c * CHUNK
            return [
                pltpu.async_copy(x_hbm.at[pl.ds(off, CHUNK)], rows[b], sem_x[b]),
                pltpu.async_copy(ia_hbm.at[pl.ds(off, CHUNK)], ia_v[b], sem_x[b]),
                pltpu.async_copy(ib_hbm.at[pl.ds(off, CHUNK)], ib_v[b], sem_x[b]),
            ]

        h_x = [None, None]
        h_s = [[], []]
        for c in range(_NCH_D):
            b = c & 1
            if c == 0:
                h_x[0] = start_loads(0, 0)
            if c + 1 < _NCH_D:
                nb = (c + 1) & 1
                for h in h_s[nb]:
                    h.wait()
                h_s[nb] = []
                h_x[nb] = start_loads(c + 1, nb)
            for h in h_s[b]:
                h.wait()
            h_s[b] = []
            for h in h_x[b]:
                h.wait()
            h_s[b] = [
                pltpu.async_copy(rows[b], buf_hbm.at[ia_v[b]], sem_s[b]),
                pltpu.async_copy(rows[b], buf_hbm.at[ib_v[b]], sem_s[b]),
            ]
        for hs in h_s:
            for h in hs:
                h.wait()

    @functools.partial(
        pl.kernel,
        mesh=mesh,
        out_type=(
            jax.ShapeDtypeStruct((T, D), jnp.float32),
            jax.ShapeDtypeStruct((T, D), jnp.float32),
        ),
        scratch_types=[
            pltpu.VMEM((BPW,), jnp.int32),
            pltpu.VMEM((BPW,), jnp.int32),
            [pltpu.VMEM((CHUNK, D), jnp.float32)] * 2,
            [pltpu.SemaphoreType.DMA] * 2,
            [pltpu.SemaphoreType.DMA] * 2,
        ],
    )
    def gather_back(ob_hbm, ia_hbm, ib_hbm, ya_hbm, yb_hbm,
                    ia_all, ib_all, rows, sem_g, sem_w):
        wid = lax.axis_index("s") * NC + lax.axis_index("c")
        base = wid * BPW
        pltpu.sync_copy(ia_hbm.at[pl.ds(base, BPW)], ia_all)
        pltpu.sync_copy(ib_hbm.at[pl.ds(base, BPW)], ib_all)

        def chunk_args(c):
            idx_all, y_hbm = (ia_all, ya_hbm) if c % 2 == 0 else (ib_all, yb_hbm)
            off = (c // 2) * CHUNK
            return idx_all.at[pl.ds(off, CHUNK)], y_hbm.at[pl.ds(base + off, CHUNK)]

        n = 2 * _NCH_D
        h_g = [None, None]
        h_w = [None, None]
        for c in range(n):
            b = c & 1
            if c == 0:
                src, _ = chunk_args(0)
                h_g[0] = pltpu.async_copy(ob_hbm.at[src], rows[0], sem_g[0])
            if c + 1 < n:
                nb = (c + 1) & 1
                if h_w[nb] is not None:
                    h_w[nb].wait()
                    h_w[nb] = None
                src, _ = chunk_args(c + 1)
                h_g[nb] = pltpu.async_copy(ob_hbm.at[src], rows[nb], sem_g[nb])
            h_g[b].wait()
            _, dst = chunk_args(c)
            h_w[b] = pltpu.async_copy(rows[b], dst, sem_w[b])
        for h in h_w:
            if h is not None:
                h.wait()

    return dispatch, gather_back


def _gemm_body(cnt_ref, buf_ref, w1_ref, w2_ref, out_ref):
    e = pl.program_id(0)
    cnt = cnt_ref[e]
    w1 = w1_ref[0]
    w2 = w2_ref[0]
    for rb in range(NB):
        @pl.when(rb * BM < cnt)
        def _():
            a = buf_ref[pl.ds(rb * BM, BM), :].astype(jnp.bfloat16)
            h = jnp.dot(a, w1, preferred_element_type=jnp.float32)
            h = jax.nn.gelu(h, approximate=True)
            out_ref[pl.ds(rb * BM, BM), :] = jnp.dot(
                h.astype(jnp.bfloat16), w2,
                preferred_element_type=jnp.float32)


_gemm = pl.pallas_call(
    _gemm_body,
    grid=(E,),
    in_specs=[
        pl.BlockSpec(memory_space=pltpu.MemorySpace.SMEM),
        pl.BlockSpec((CAP, D), lambda e: (e, 0)),
        pl.BlockSpec((1, D, FF), lambda e: (e, 0, 0)),
        pl.BlockSpec((1, FF, D), lambda e: (e, 0, 0)),
    ],
    out_specs=pl.BlockSpec((CAP, D), lambda e: (e, 0)),
    out_shape=jax.ShapeDtypeStruct((BUF_ROWS, D), jnp.float32),
    compiler_params=pltpu.CompilerParams(
        dimension_semantics=("parallel",),
        allow_input_fusion=[False, False, True, True],
        vmem_limit_bytes=100 * 1024 * 1024),
)


def _combine_body(ya_ref, yb_ref, wa_ref, wb_ref, o_ref):
    wa = wa_ref[...]
    wb = wb_ref[...]
    o_ref[...] = (jnp.where(wa != 0.0, wa * ya_ref[...], 0.0)
                  + jnp.where(wb != 0.0, wb * yb_ref[...], 0.0))


_CB = 512
_combine = pl.pallas_call(
    _combine_body,
    grid=(T // _CB,),
    in_specs=[
        pl.BlockSpec((_CB, D), lambda i: (i, 0)),
        pl.BlockSpec((_CB, D), lambda i: (i, 0)),
        pl.BlockSpec((_CB, 1), lambda i: (i, 0)),
        pl.BlockSpec((_CB, 1), lambda i: (i, 0)),
    ],
    out_specs=pl.BlockSpec((_CB, D), lambda i: (i, 0)),
    out_shape=jax.ShapeDtypeStruct((T, D), jnp.float32),
)


def kernel(x, scores, expert_weights, top_experts, W1, W2):
    del scores
    te128 = top_experts.astype(jnp.int32).reshape(64, 128)
    w128 = expert_weights.astype(jnp.float32).reshape(64, 128)
    slot, wm, cnt = _routing(te128, w128)

    slot2 = slot.reshape(T, K)
    ia = slot2[:, 0]
    ib = slot2[:, 1]
    wm2 = wm.reshape(T, K)
    wa = wm2[:, 0:1]
    wb = wm2[:, 1:2]
    counts = cnt[:, 0]

    dispatch, gather_back = _sc_kernels()
    buf = dispatch(x, ia, ib)
    out_buf = _gemm(counts, buf, W1.astype(jnp.bfloat16),
                    W2.astype(jnp.bfloat16))
    ya, yb = gather_back(out_buf, ia, ib)
    return _combine(ya, yb, wa, wb)
